# Initial kernel scaffold; baseline (speedup 1.0000x reference)
import jax
import jax.numpy as jnp
from jax.experimental import pallas as pl


def kernel(x, edge_index, W1, b1, W2, b2, W3, b3, H1, hb1, H2, hb2):
    raise NotImplementedError("write your pallas kernel here")


# speedup vs baseline: 14.2953x; 14.2953x over previous
import functools

import jax
import jax.numpy as jnp
from jax import lax
from jax.experimental import pallas as pl
from jax.experimental.pallas import tpu as pltpu
from jax.experimental.pallas import tpu_sc as plsc

N = 10000
NP = 10240
JUNK = 10000
D_IN = 128
HID = 64
E = 320000
NUM_TILES = 32
CHUNK = 128
CPT = 80
E_PAD = NUM_TILES * CPT * CHUNK
ROWS_PT = NP // 16

_mesh = plsc.VectorSubcoreMesh(core_axis_name="c", subcore_axis_name="s")


def _sc_degree(dst3):
    @functools.partial(
        pl.kernel,
        out_type=jax.ShapeDtypeStruct((2, NP), jnp.float32),
        mesh=_mesh,
        scratch_types=[
            pltpu.VMEM((CPT, CHUNK), jnp.int32),
            pltpu.VMEM((CHUNK,), jnp.float32),
            pltpu.VMEM((ROWS_PT,), jnp.float32),
            pltpu.VMEM_SHARED((NP,), jnp.float32),
        ],
    )
    def deg_kernel(dst_hbm, out_hbm, dst_all, ones_v, zero_v, acc_sh):
        c = lax.axis_index("c")
        s = lax.axis_index("s")
        wid = c * 16 + s

        @pl.loop(0, CHUNK, step=16)
        def _(i):
            ones_v[pl.ds(i, 16)] = jnp.ones((16,), jnp.float32)

        @pl.loop(0, ROWS_PT, step=16)
        def _(i):
            zero_v[pl.ds(i, 16)] = jnp.zeros((16,), jnp.float32)

        pltpu.sync_copy(zero_v, acc_sh.at[pl.ds(s * ROWS_PT, ROWS_PT)])
        pltpu.sync_copy(dst_hbm.at[wid], dst_all)
        plsc.subcore_barrier()

        @pl.loop(0, CPT)
        def _(j):
            pltpu.sync_copy(ones_v, acc_sh.at[dst_all.at[j]], add=True)

        plsc.subcore_barrier()
        pltpu.sync_copy(acc_sh.at[pl.ds(s * ROWS_PT, ROWS_PT)],
                        out_hbm.at[c, pl.ds(s * ROWS_PT, ROWS_PT)])

    return deg_kernel(dst3)


def _sc_propagate(y, src3, dst3):
    @functools.partial(
        pl.kernel,
        out_type=jax.ShapeDtypeStruct((2, NP, HID), jnp.float32),
        mesh=_mesh,
        compiler_params=pltpu.CompilerParams(use_tc_tiling_on_sc=False),
        scratch_types=[
            pltpu.VMEM((CPT, CHUNK), jnp.int32),
            pltpu.VMEM((CPT, CHUNK), jnp.int32),
            pltpu.VMEM((CHUNK, HID), jnp.float32),
            pltpu.VMEM((CHUNK, HID), jnp.float32),
            pltpu.VMEM_SHARED((NP, HID), jnp.float32),
            pltpu.SemaphoreType.DMA,
            pltpu.SemaphoreType.DMA,
        ],
    )
    def prop_kernel(y_hbm, src_hbm, dst_hbm, out_hbm,
                    src_all, dst_all, rows0, rows1, acc_sh, sem0, sem1):
        c = lax.axis_index("c")
        s = lax.axis_index("s")
        wid = c * 16 + s

        @pl.loop(0, CHUNK)
        def _(i):
            for k in range(HID // 16):
                rows0[i, pl.ds(k * 16, 16)] = jnp.zeros((16,), jnp.float32)

        for t in range(ROWS_PT // CHUNK):
            pltpu.sync_copy(
                rows0, acc_sh.at[pl.ds(s * ROWS_PT + t * CHUNK, CHUNK)])

        pltpu.sync_copy(src_hbm.at[wid], src_all)
        pltpu.sync_copy(dst_hbm.at[wid], dst_all)
        plsc.subcore_barrier()

        pltpu.async_copy(y_hbm.at[src_all.at[0]], rows0, sem0)

        @pl.loop(0, CPT // 2)
        def _(i):
            j0 = i * 2
            j1 = j0 + 1
            jn = lax.rem(j0 + 2, CPT)
            pltpu.async_copy(y_hbm.at[src_all.at[j1]], rows1, sem1)
            pltpu.make_async_copy(y_hbm.at[src_all.at[j0]], rows0, sem0).wait()
            pltpu.sync_copy(rows0, acc_sh.at[dst_all.at[j0]], add=True)
            pltpu.async_copy(y_hbm.at[src_all.at[jn]], rows0, sem0)
            pltpu.make_async_copy(y_hbm.at[src_all.at[j1]], rows1, sem1).wait()
            pltpu.sync_copy(rows1, acc_sh.at[dst_all.at[j1]], add=True)

        pltpu.make_async_copy(y_hbm.at[src_all.at[0]], rows0, sem0).wait()

        plsc.subcore_barrier()
        pltpu.sync_copy(acc_sh.at[pl.ds(s * ROWS_PT, ROWS_PT)],
                        out_hbm.at[c, pl.ds(s * ROWS_PT, ROWS_PT)])

    return prop_kernel(y, src3, dst3)


RB = 1280
GRID = NP // RB


def _dinv_of(degp_blk):
    deg = degp_blk[0, :] + degp_blk[1, :] + 1.0
    return lax.rsqrt(deg)


def _tc_first(x_pad, w1, degp):
    def body(x_ref, w_ref, degp_ref, y_ref):
        dinv = _dinv_of(degp_ref)
        y = jnp.dot(x_ref[...], w_ref[...], preferred_element_type=jnp.float32)
        y_ref[...] = y * dinv[:, None]

    return pl.pallas_call(
        body,
        grid=(GRID,),
        in_specs=[
            pl.BlockSpec((RB, D_IN), lambda i: (i, 0)),
            pl.BlockSpec((D_IN, HID), lambda i: (0, 0)),
            pl.BlockSpec((2, RB), lambda i: (0, i)),
        ],
        out_specs=pl.BlockSpec((RB, HID), lambda i: (i, 0)),
        out_shape=jax.ShapeDtypeStruct((NP, HID), jnp.float32),
    )(x_pad, w1, degp)


def _tc_mid(p, y_prev, degp, b, w_next):
    def body(p_ref, y_ref, degp_ref, b_ref, w_ref, o_ref):
        dinv = _dinv_of(degp_ref)
        ssum = p_ref[0] + p_ref[1] + y_ref[...]
        h = jnp.maximum(ssum * dinv[:, None] + b_ref[...][None, :], 0.0)
        y = jnp.dot(h, w_ref[...], preferred_element_type=jnp.float32)
        o_ref[...] = y * dinv[:, None]

    return pl.pallas_call(
        body,
        grid=(GRID,),
        in_specs=[
            pl.BlockSpec((2, RB, HID), lambda i: (0, i, 0)),
            pl.BlockSpec((RB, HID), lambda i: (i, 0)),
            pl.BlockSpec((2, RB), lambda i: (0, i)),
            pl.BlockSpec((HID,), lambda i: (0,)),
            pl.BlockSpec((HID, HID), lambda i: (0, 0)),
        ],
        out_specs=pl.BlockSpec((RB, HID), lambda i: (i, 0)),
        out_shape=jax.ShapeDtypeStruct((NP, HID), jnp.float32),
    )(p, y_prev, degp, b, w_next)


def _tc_head(p, y_prev, degp, b3, h1, hb1, h2, hb2):
    def body(p_ref, y_ref, degp_ref, b_ref, h1_ref, hb1_ref, h2_ref, hb2_ref,
             o_ref):
        dinv = _dinv_of(degp_ref)
        ssum = p_ref[0] + p_ref[1] + y_ref[...]
        h = jnp.maximum(ssum * dinv[:, None] + b_ref[...][None, :], 0.0)
        g = jnp.dot(h, h1_ref[...], preferred_element_type=jnp.float32)
        g = jnp.maximum(g + hb1_ref[...][None, :], 0.0)
        o = jnp.dot(g, h2_ref[...], preferred_element_type=jnp.float32)
        o_ref[...] = o + hb2_ref[...][None, :]

    return pl.pallas_call(
        body,
        grid=(GRID,),
        in_specs=[
            pl.BlockSpec((2, RB, HID), lambda i: (0, i, 0)),
            pl.BlockSpec((RB, HID), lambda i: (i, 0)),
            pl.BlockSpec((2, RB), lambda i: (0, i)),
            pl.BlockSpec((HID,), lambda i: (0,)),
            pl.BlockSpec((HID, HID // 2), lambda i: (0, 0)),
            pl.BlockSpec((HID // 2,), lambda i: (0,)),
            pl.BlockSpec((HID // 2, 2), lambda i: (0, 0)),
            pl.BlockSpec((2,), lambda i: (0,)),
        ],
        out_specs=pl.BlockSpec((RB, 2), lambda i: (i, 0)),
        out_shape=jax.ShapeDtypeStruct((NP, 2), jnp.float32),
    )(p, y_prev, degp, b3, h1, hb1, h2, hb2)


def kernel(x, edge_index, W1, b1, W2, b2, W3, b3, H1, hb1, H2, hb2):
    ei = edge_index.astype(jnp.int32)
    pad = jnp.full((E_PAD - E,), JUNK, jnp.int32)
    src3 = jnp.concatenate([ei[0], pad]).reshape(NUM_TILES, CPT, CHUNK)
    dst3 = jnp.concatenate([ei[1], pad]).reshape(NUM_TILES, CPT, CHUNK)
    x_pad = jnp.pad(x.astype(jnp.float32), ((0, NP - N), (0, 0)))

    degp = _sc_degree(dst3)
    y1 = _tc_first(x_pad, W1, degp)
    p1 = _sc_propagate(y1, src3, dst3)
    y2 = _tc_mid(p1, y1, degp, b1, W2)
    p2 = _sc_propagate(y2, src3, dst3)
    y3 = _tc_mid(p2, y2, degp, b2, W3)
    p3 = _sc_propagate(y3, src3, dst3)
    out = _tc_head(p3, y3, degp, b3, H1, hb1, H2, hb2)
    return out[:N]


# speedup vs baseline: 17.4236x; 1.2188x over previous
import functools

import jax
import jax.numpy as jnp
from jax import lax
from jax.experimental import pallas as pl
from jax.experimental.pallas import tpu as pltpu
from jax.experimental.pallas import tpu_sc as plsc

N = 10000
NP = 10240
JUNK = 10000
D_IN = 128
HID = 64
E = 320000
CHUNK = 128
CPT = 80
NBUF = 5
GLAG = 4
SLAG = 1
E_PAD = 32 * CPT * CHUNK
HALF = 5000
YROWS = 5008
ACC_ROWS = 10016
ACC_PT = ACC_ROWS // 16

_mesh = plsc.VectorSubcoreMesh(core_axis_name="c", subcore_axis_name="s")


def _sc_degree(dst3):
    @functools.partial(
        pl.kernel,
        out_type=jax.ShapeDtypeStruct((2, NP), jnp.float32),
        mesh=_mesh,
        scratch_types=[
            pltpu.VMEM((CPT, CHUNK), jnp.int32),
            pltpu.VMEM((CHUNK,), jnp.float32),
            pltpu.VMEM((NP // 16,), jnp.float32),
            pltpu.VMEM_SHARED((NP,), jnp.float32),
        ],
    )
    def deg_kernel(dst_hbm, out_hbm, dst_all, ones_v, zero_v, acc_sh):
        c = lax.axis_index("c")
        s = lax.axis_index("s")
        wid = c * 16 + s
        rpt = NP // 16

        @pl.loop(0, CHUNK, step=16)
        def _(i):
            ones_v[pl.ds(i, 16)] = jnp.ones((16,), jnp.float32)

        @pl.loop(0, rpt, step=16)
        def _(i):
            zero_v[pl.ds(i, 16)] = jnp.zeros((16,), jnp.float32)

        pltpu.sync_copy(zero_v, acc_sh.at[pl.ds(s * rpt, rpt)])
        pltpu.sync_copy(dst_hbm.at[wid], dst_all)
        plsc.subcore_barrier()

        @pl.loop(0, CPT)
        def _(j):
            pltpu.sync_copy(ones_v, acc_sh.at[dst_all.at[j]], add=True)

        plsc.subcore_barrier()
        pltpu.sync_copy(acc_sh.at[pl.ds(s * rpt, rpt)],
                        out_hbm.at[c, pl.ds(s * rpt, rpt)])

    return deg_kernel(dst3)


def _make_propagate():
    @functools.partial(
        pl.kernel,
        out_type=jax.ShapeDtypeStruct((2, ACC_ROWS, HID), jnp.float32),
        mesh=_mesh,
        compiler_params=pltpu.CompilerParams(use_tc_tiling_on_sc=False),
        scratch_types=[
            pltpu.VMEM((CPT * CHUNK,), jnp.int32),
            pltpu.VMEM((CPT, CHUNK), jnp.int32),
            [pltpu.VMEM((CHUNK, HID), jnp.float32) for _ in range(NBUF)],
            [pltpu.VMEM((CHUNK,), jnp.int32) for _ in range(NBUF)],
            [pltpu.VMEM((CHUNK,), jnp.int32) for _ in range(NBUF)],
            pltpu.VMEM((CHUNK, HID), jnp.float32),
            pltpu.VMEM_SHARED((YROWS, HID), jnp.float32),
            pltpu.VMEM_SHARED((ACC_ROWS, HID), jnp.float32),
            [pltpu.SemaphoreType.DMA for _ in range(NBUF)],
            [pltpu.SemaphoreType.DMA for _ in range(NBUF)],
        ],
    )
    def prop_kernel(y_hbm, src_hbm, dst_hbm, out_hbm,
                    src_all, dst_all, bufs, srcr, dstr, zbuf, y_sh, acc_sh,
                    gsem, ssem):
        c = lax.axis_index("c")
        s = lax.axis_index("s")
        wid = c * 16 + s

        def prep(g, b, pbase):
            for k in range(CHUNK // 16):
                sl = pl.ds(k * 16, 16)
                v = src_all[pl.ds(g * CHUNK + k * 16, 16)] - pbase
                ok = (v >= 0) & (v < HALF)
                srcr[b][sl] = jnp.where(ok, v, HALF)
                dstr[b][sl] = dst_all[g, sl]

        def gather(b):
            return pltpu.make_async_copy(y_sh.at[srcr[b]], bufs[b], gsem[b])

        def scatter_start(b):
            pltpu.async_copy(bufs[b], acc_sh.at[dstr[b]], ssem[b], add=True)

        def scatter_wait(b):
            pltpu.make_async_copy(bufs[b], acc_sh.at[dstr[b]], ssem[b]).wait()

        @pl.loop(0, CHUNK)
        def _(i):
            for k in range(HID // 16):
                zbuf[i, pl.ds(k * 16, 16)] = jnp.zeros((16,), jnp.float32)

        for off in range(0, 512, CHUNK):
            pltpu.sync_copy(zbuf, acc_sh.at[pl.ds(s * ACC_PT + off, CHUNK)])
        pltpu.sync_copy(zbuf.at[pl.ds(0, ACC_PT - 512)],
                        acc_sh.at[pl.ds(s * ACC_PT + 512, ACC_PT - 512)])

        @pl.when(s == 0)
        def _():
            pltpu.sync_copy(zbuf.at[pl.ds(0, YROWS - HALF)],
                            y_sh.at[pl.ds(HALF, YROWS - HALF)])

        pltpu.sync_copy(src_hbm.at[wid], src_all)
        pltpu.sync_copy(dst_hbm.at[wid], dst_all)

        for p in range(2):
            pbase = p * HALF
            @pl.when(s < 8)
            def _():
                pltpu.sync_copy(y_hbm.at[pl.ds(pbase + s * 625, 625)],
                                y_sh.at[pl.ds(s * 625, 625)])
            plsc.subcore_barrier()

            for g0 in range(GLAG):
                prep(g0, g0 % NBUF, pbase)
                gather(g0 % NBUF).start()

            @pl.loop(0, CPT // NBUF)
            def _(i):
                for b in range(NBUF):
                    g = i * NBUF + b

                    @pl.when(g >= SLAG)
                    def _():
                        scatter_wait((b - SLAG) % NBUF)

                    @pl.when(g + GLAG < CPT)
                    def _():
                        bn = (b + GLAG) % NBUF
                        prep(g + GLAG, bn, pbase)
                        gather(bn).start()

                    gather(b).wait()
                    scatter_start(b)

            for g in range(CPT - SLAG, CPT):
                scatter_wait(g % NBUF)
            plsc.subcore_barrier()

        pltpu.sync_copy(acc_sh.at[pl.ds(s * ACC_PT, ACC_PT)],
                        out_hbm.at[c, pl.ds(s * ACC_PT, ACC_PT)])

    return prop_kernel


_sc_propagate = _make_propagate()


RB = 1280
GRID = NP // RB


def _dinv_of(degp_blk):
    deg = degp_blk[0, :] + degp_blk[1, :] + 1.0
    return lax.rsqrt(deg)


def _combine(p_ref):
    ssum = p_ref[0, :N] + p_ref[1, :N]
    return jnp.concatenate(
        [ssum, jnp.zeros((NP - N, HID), jnp.float32)], axis=0)


def _tc_first(x_pad, w1, degp):
    def body(x_ref, w_ref, degp_ref, y_ref):
        dinv = _dinv_of(degp_ref)
        y = jnp.dot(x_ref[...], w_ref[...], preferred_element_type=jnp.float32)
        y_ref[...] = y * dinv[:, None]

    return pl.pallas_call(
        body,
        grid=(GRID,),
        in_specs=[
            pl.BlockSpec((RB, D_IN), lambda i: (i, 0)),
            pl.BlockSpec((D_IN, HID), lambda i: (0, 0)),
            pl.BlockSpec((2, RB), lambda i: (0, i)),
        ],
        out_specs=pl.BlockSpec((RB, HID), lambda i: (i, 0)),
        out_shape=jax.ShapeDtypeStruct((NP, HID), jnp.float32),
    )(x_pad, w1, degp)


def _tc_mid(p, y_prev, degp, b, w_next):
    def body(p_ref, y_ref, degp_ref, b_ref, w_ref, o_ref):
        dinv = _dinv_of(degp_ref)
        ssum = _combine(p_ref) + y_ref[...]
        h = jnp.maximum(ssum * dinv[:, None] + b_ref[...][None, :], 0.0)
        y = jnp.dot(h, w_ref[...], preferred_element_type=jnp.float32)
        o_ref[...] = y * dinv[:, None]

    return pl.pallas_call(
        body,
        out_shape=jax.ShapeDtypeStruct((NP, HID), jnp.float32),
    )(p, y_prev, degp, b, w_next)


def _tc_head(p, y_prev, degp, b3, h1, hb1, h2, hb2):
    def body(p_ref, y_ref, degp_ref, b_ref, h1_ref, hb1_ref, h2_ref, hb2_ref,
             o_ref):
        dinv = _dinv_of(degp_ref)
        ssum = _combine(p_ref) + y_ref[...]
        h = jnp.maximum(ssum * dinv[:, None] + b_ref[...][None, :], 0.0)
        g = jnp.dot(h, h1_ref[...], preferred_element_type=jnp.float32)
        g = jnp.maximum(g + hb1_ref[...][None, :], 0.0)
        o = jnp.dot(g, h2_ref[...], preferred_element_type=jnp.float32)
        o_ref[...] = o + hb2_ref[...][None, :]

    return pl.pallas_call(
        body,
        out_shape=jax.ShapeDtypeStruct((NP, 2), jnp.float32),
    )(p, y_prev, degp, b3, h1, hb1, h2, hb2)


def kernel(x, edge_index, W1, b1, W2, b2, W3, b3, H1, hb1, H2, hb2):
    ei = edge_index.astype(jnp.int32)
    src3 = jnp.concatenate(
        [ei[0], jnp.zeros((E_PAD - E,), jnp.int32)]).reshape(32, CPT * CHUNK)
    dst3 = jnp.concatenate(
        [ei[1], jnp.full((E_PAD - E,), JUNK, jnp.int32)]
    ).reshape(32, CPT, CHUNK)
    x_pad = jnp.pad(x.astype(jnp.float32), ((0, NP - N), (0, 0)))

    degp = _sc_degree(dst3)
    y1 = _tc_first(x_pad, W1, degp)
    p1 = _sc_propagate(y1, src3, dst3)
    y2 = _tc_mid(p1, y1, degp, b1, W2)
    p2 = _sc_propagate(y2, src3, dst3)
    y3 = _tc_mid(p2, y2, degp, b2, W3)
    p3 = _sc_propagate(y3, src3, dst3)
    out = _tc_head(p3, y3, degp, b3, H1, hb1, H2, hb2)
    return out[:N]


# speedup vs baseline: 17.4774x; 1.0031x over previous
import functools

import jax
import jax.numpy as jnp
from jax import lax
from jax.experimental import pallas as pl
from jax.experimental.pallas import tpu as pltpu
from jax.experimental.pallas import tpu_sc as plsc

N = 10000
NP = 10240
JUNK = 10000
D_IN = 128
HID = 64
E = 320000
CHUNK = 128
CPT = 80
NBUF = 5
GLAG = 4
SLAG = 1
E_PAD = 32 * CPT * CHUNK
HALF = 5000
YROWS = 5008
ACC_ROWS = 10016
ACC_PT = ACC_ROWS // 16

_mesh = plsc.VectorSubcoreMesh(core_axis_name="c", subcore_axis_name="s")


def _sc_degree(dst3):
    @functools.partial(
        pl.kernel,
        out_type=jax.ShapeDtypeStruct((2, NP), jnp.float32),
        mesh=_mesh,
        scratch_types=[
            pltpu.VMEM((CPT, CHUNK), jnp.int32),
            pltpu.VMEM((CHUNK,), jnp.float32),
            pltpu.VMEM((NP // 16,), jnp.float32),
            pltpu.VMEM_SHARED((NP,), jnp.float32),
        ],
    )
    def deg_kernel(dst_hbm, out_hbm, dst_all, ones_v, zero_v, acc_sh):
        c = lax.axis_index("c")
        s = lax.axis_index("s")
        wid = c * 16 + s
        rpt = NP // 16

        @pl.loop(0, CHUNK, step=16)
        def _(i):
            ones_v[pl.ds(i, 16)] = jnp.ones((16,), jnp.float32)

        @pl.loop(0, rpt, step=16)
        def _(i):
            zero_v[pl.ds(i, 16)] = jnp.zeros((16,), jnp.float32)

        pltpu.sync_copy(zero_v, acc_sh.at[pl.ds(s * rpt, rpt)])
        pltpu.sync_copy(dst_hbm.at[wid], dst_all)
        plsc.subcore_barrier()

        @pl.loop(0, CPT)
        def _(j):
            pltpu.sync_copy(ones_v, acc_sh.at[dst_all.at[j]], add=True)

        plsc.subcore_barrier()
        pltpu.sync_copy(acc_sh.at[pl.ds(s * rpt, rpt)],
                        out_hbm.at[c, pl.ds(s * rpt, rpt)])

    return deg_kernel(dst3)


def _make_propagate():
    @functools.partial(
        pl.kernel,
        out_type=jax.ShapeDtypeStruct((2, ACC_ROWS, HID), jnp.float32),
        mesh=_mesh,
        compiler_params=pltpu.CompilerParams(use_tc_tiling_on_sc=False),
        scratch_types=[
            pltpu.VMEM((CPT * CHUNK,), jnp.int32),
            pltpu.VMEM((CPT, CHUNK), jnp.int32),
            [pltpu.VMEM((CHUNK, HID), jnp.float32) for _ in range(NBUF)],
            [pltpu.VMEM((CHUNK,), jnp.int32) for _ in range(NBUF)],
            [pltpu.VMEM((CHUNK,), jnp.int32) for _ in range(NBUF)],
            pltpu.VMEM((CHUNK, HID), jnp.float32),
            pltpu.VMEM_SHARED((YROWS, HID), jnp.float32),
            pltpu.VMEM_SHARED((ACC_ROWS, HID), jnp.float32),
            [pltpu.SemaphoreType.DMA for _ in range(NBUF)],
            [pltpu.SemaphoreType.DMA for _ in range(NBUF)],
        ],
    )
    def prop_kernel(y_hbm, src_hbm, dst_hbm, out_hbm,
                    src_all, dst_all, bufs, srcr, dstr, zbuf, y_sh, acc_sh,
                    gsem, ssem):
        c = lax.axis_index("c")
        s = lax.axis_index("s")
        wid = c * 16 + s

        def prep(g, b, pbase):
            for k in range(CHUNK // 16):
                sl = pl.ds(k * 16, 16)
                v = src_all[pl.ds(g * CHUNK + k * 16, 16)] - pbase
                ok = (v >= 0) & (v < HALF)
                srcr[b][sl] = jnp.where(ok, v, HALF)
                dstr[b][sl] = dst_all[g, sl]

        def gather(b):
            return pltpu.make_async_copy(y_sh.at[srcr[b]], bufs[b], gsem[b])

        def scatter_start(b):
            pltpu.async_copy(bufs[b], acc_sh.at[dstr[b]], ssem[b], add=True)

        def scatter_wait(b):
            pltpu.make_async_copy(bufs[b], acc_sh.at[dstr[b]], ssem[b]).wait()

        @pl.loop(0, CHUNK)
        def _(i):
            for k in range(HID // 16):
                zbuf[i, pl.ds(k * 16, 16)] = jnp.zeros((16,), jnp.float32)

        for off in range(0, 512, CHUNK):
            pltpu.sync_copy(zbuf, acc_sh.at[pl.ds(s * ACC_PT + off, CHUNK)])
        pltpu.sync_copy(zbuf.at[pl.ds(0, ACC_PT - 512)],
                        acc_sh.at[pl.ds(s * ACC_PT + 512, ACC_PT - 512)])

        @pl.when(s == 0)
        def _():
            pltpu.sync_copy(zbuf.at[pl.ds(0, YROWS - HALF)],
                            y_sh.at[pl.ds(HALF, YROWS - HALF)])

        pltpu.sync_copy(src_hbm.at[wid], src_all)
        pltpu.sync_copy(dst_hbm.at[wid], dst_all)

        for p in range(2):
            pbase = p * HALF
            @pl.when(s < 8)
            def _():
                pltpu.sync_copy(y_hbm.at[pl.ds(pbase + s * 625, 625)],
                                y_sh.at[pl.ds(s * 625, 625)])
            plsc.subcore_barrier()

            for g0 in range(GLAG):
                prep(g0, g0 % NBUF, pbase)
                gather(g0 % NBUF).start()

            @pl.loop(0, CPT // NBUF)
            def _(i):
                for b in range(NBUF):
                    g = i * NBUF + b

                    @pl.when(g >= SLAG)
                    def _():
                        scatter_wait((b - SLAG) % NBUF)

                    @pl.when(g + GLAG < CPT)
                    def _():
                        bn = (b + GLAG) % NBUF
                        prep(g + GLAG, bn, pbase)
                        gather(bn).start()

                    gather(b).wait()
                    scatter_start(b)

            for g in range(CPT - SLAG, CPT):
                scatter_wait(g % NBUF)
            plsc.subcore_barrier()

        pltpu.sync_copy(acc_sh.at[pl.ds(s * ACC_PT, ACC_PT)],
                        out_hbm.at[c, pl.ds(s * ACC_PT, ACC_PT)])

    return prop_kernel


_sc_propagate = _make_propagate()


RB = 1280
GRID = NP // RB


def _dinv_of(degp_blk):
    deg = degp_blk[0, :] + degp_blk[1, :] + 1.0
    return lax.rsqrt(deg)


def _combine(p_ref):
    ssum = p_ref[0, :N] + p_ref[1, :N]
    return jnp.concatenate(
        [ssum, jnp.zeros((NP - N, HID), jnp.float32)], axis=0)


def _tc_matmul1(x_pad, w1):
    def body(x_ref, w_ref, z_ref):
        z_ref[...] = jnp.dot(x_ref[...], w_ref[...],
                             preferred_element_type=jnp.float32)

    return pl.pallas_call(
        body,
        grid=(GRID,),
        in_specs=[
            pl.BlockSpec((RB, D_IN), lambda i: (i, 0)),
            pl.BlockSpec((D_IN, HID), lambda i: (0, 0)),
        ],
        out_specs=pl.BlockSpec((RB, HID), lambda i: (i, 0)),
        out_shape=jax.ShapeDtypeStruct((NP, HID), jnp.float32),
    )(x_pad, w1)


def _tc_scale(z1, degp):
    def body(z_ref, degp_ref, y_ref):
        dinv = _dinv_of(degp_ref)
        y_ref[...] = z_ref[...] * dinv[:, None]

    return pl.pallas_call(
        body,
        out_shape=jax.ShapeDtypeStruct((NP, HID), jnp.float32),
    )(z1, degp)


def _tc_mid(p, y_prev, degp, b, w_next):
    def body(p_ref, y_ref, degp_ref, b_ref, w_ref, o_ref):
        dinv = _dinv_of(degp_ref)
        ssum = _combine(p_ref) + y_ref[...]
        h = jnp.maximum(ssum * dinv[:, None] + b_ref[...][None, :], 0.0)
        y = jnp.dot(h, w_ref[...], preferred_element_type=jnp.float32)
        o_ref[...] = y * dinv[:, None]

    return pl.pallas_call(
        body,
        out_shape=jax.ShapeDtypeStruct((NP, HID), jnp.float32),
    )(p, y_prev, degp, b, w_next)


def _tc_head(p, y_prev, degp, b3, h1, hb1, h2, hb2):
    def body(p_ref, y_ref, degp_ref, b_ref, h1_ref, hb1_ref, h2_ref, hb2_ref,
             o_ref):
        dinv = _dinv_of(degp_ref)
        ssum = _combine(p_ref) + y_ref[...]
        h = jnp.maximum(ssum * dinv[:, None] + b_ref[...][None, :], 0.0)
        g = jnp.dot(h, h1_ref[...], preferred_element_type=jnp.float32)
        g = jnp.maximum(g + hb1_ref[...][None, :], 0.0)
        o = jnp.dot(g, h2_ref[...], preferred_element_type=jnp.float32)
        o_ref[...] = o + hb2_ref[...][None, :]

    return pl.pallas_call(
        body,
        out_shape=jax.ShapeDtypeStruct((NP, 2), jnp.float32),
    )(p, y_prev, degp, b3, h1, hb1, h2, hb2)


def kernel(x, edge_index, W1, b1, W2, b2, W3, b3, H1, hb1, H2, hb2):
    ei = edge_index.astype(jnp.int32)
    src3 = jnp.concatenate(
        [ei[0], jnp.zeros((E_PAD - E,), jnp.int32)]).reshape(32, CPT * CHUNK)
    dst3 = jnp.concatenate(
        [ei[1], jnp.full((E_PAD - E,), JUNK, jnp.int32)]
    ).reshape(32, CPT, CHUNK)
    x_pad = jnp.pad(x.astype(jnp.float32), ((0, NP - N), (0, 0)))

    degp = _sc_degree(dst3)
    z1 = _tc_matmul1(x_pad, W1)
    y1 = _tc_scale(z1, degp)
    p1 = _sc_propagate(y1, src3, dst3)
    y2 = _tc_mid(p1, y1, degp, b1, W2)
    p2 = _sc_propagate(y2, src3, dst3)
    y3 = _tc_mid(p2, y2, degp, b2, W3)
    p3 = _sc_propagate(y3, src3, dst3)
    out = _tc_head(p3, y3, degp, b3, H1, hb1, H2, hb2)
    return out[:N]


# speedup vs baseline: 17.7811x; 1.0174x over previous
import functools

import jax
import jax.numpy as jnp
from jax import lax
from jax.experimental import pallas as pl
from jax.experimental.pallas import tpu as pltpu
from jax.experimental.pallas import tpu_sc as plsc

N = 10000
NP = 10240
JUNK = 10000
D_IN = 128
HID = 64
E = 320000
CHUNK = 128
CPT = 80
NBUF = 5
GLAG = 3
SLAG = 2
E_PAD = 32 * CPT * CHUNK
HALF = 5000
YROWS = 5008
ACC_ROWS = 10016
ACC_PT = ACC_ROWS // 16

_mesh = plsc.VectorSubcoreMesh(core_axis_name="c", subcore_axis_name="s")


def _sc_degree(dst3):
    @functools.partial(
        pl.kernel,
        out_type=jax.ShapeDtypeStruct((2, NP), jnp.float32),
        mesh=_mesh,
        scratch_types=[
            pltpu.VMEM((CPT, CHUNK), jnp.int32),
            pltpu.VMEM((CHUNK,), jnp.float32),
            pltpu.VMEM((NP // 16,), jnp.float32),
            pltpu.VMEM_SHARED((NP,), jnp.float32),
        ],
    )
    def deg_kernel(dst_hbm, out_hbm, dst_all, ones_v, zero_v, acc_sh):
        c = lax.axis_index("c")
        s = lax.axis_index("s")
        wid = c * 16 + s
        rpt = NP // 16

        @pl.loop(0, CHUNK, step=16)
        def _(i):
            ones_v[pl.ds(i, 16)] = jnp.ones((16,), jnp.float32)

        @pl.loop(0, rpt, step=16)
        def _(i):
            zero_v[pl.ds(i, 16)] = jnp.zeros((16,), jnp.float32)

        pltpu.sync_copy(zero_v, acc_sh.at[pl.ds(s * rpt, rpt)])
        pltpu.sync_copy(dst_hbm.at[wid], dst_all)
        plsc.subcore_barrier()

        @pl.loop(0, CPT)
        def _(j):
            pltpu.sync_copy(ones_v, acc_sh.at[dst_all.at[j]], add=True)

        plsc.subcore_barrier()
        pltpu.sync_copy(acc_sh.at[pl.ds(s * rpt, rpt)],
                        out_hbm.at[c, pl.ds(s * rpt, rpt)])

    return deg_kernel(dst3)


def _make_propagate():
    @functools.partial(
        pl.kernel,
        out_type=jax.ShapeDtypeStruct((2, ACC_ROWS, HID), jnp.float32),
        mesh=_mesh,
        compiler_params=pltpu.CompilerParams(use_tc_tiling_on_sc=False),
        scratch_types=[
            pltpu.VMEM((CPT * CHUNK,), jnp.int32),
            pltpu.VMEM((CPT, CHUNK), jnp.int32),
            [pltpu.VMEM((CHUNK, HID), jnp.float32) for _ in range(NBUF)],
            [pltpu.VMEM((CHUNK,), jnp.int32) for _ in range(NBUF)],
            [pltpu.VMEM((CHUNK,), jnp.int32) for _ in range(NBUF)],
            pltpu.VMEM((CHUNK, HID), jnp.float32),
            pltpu.VMEM_SHARED((YROWS, HID), jnp.float32),
            pltpu.VMEM_SHARED((ACC_ROWS, HID), jnp.float32),
            [pltpu.SemaphoreType.DMA for _ in range(NBUF)],
            [pltpu.SemaphoreType.DMA for _ in range(NBUF)],
        ],
    )
    def prop_kernel(y_hbm, src_hbm, dst_hbm, out_hbm,
                    src_all, dst_all, bufs, srcr, dstr, zbuf, y_sh, acc_sh,
                    gsem, ssem):
        c = lax.axis_index("c")
        s = lax.axis_index("s")
        wid = c * 16 + s

        def prep(g, b, pbase):
            for k in range(CHUNK // 16):
                sl = pl.ds(k * 16, 16)
                v = src_all[pl.ds(g * CHUNK + k * 16, 16)] - pbase
                ok = (v >= 0) & (v < HALF)
                srcr[b][sl] = jnp.where(ok, v, HALF)
                dstr[b][sl] = dst_all[g, sl]

        def gather(b):
            return pltpu.make_async_copy(y_sh.at[srcr[b]], bufs[b], gsem[b])

        def scatter_start(b):
            pltpu.async_copy(bufs[b], acc_sh.at[dstr[b]], ssem[b], add=True)

        def scatter_wait(b):
            pltpu.make_async_copy(bufs[b], acc_sh.at[dstr[b]], ssem[b]).wait()

        @pl.loop(0, CHUNK)
        def _(i):
            for k in range(HID // 16):
                zbuf[i, pl.ds(k * 16, 16)] = jnp.zeros((16,), jnp.float32)

        for off in range(0, 512, CHUNK):
            pltpu.sync_copy(zbuf, acc_sh.at[pl.ds(s * ACC_PT + off, CHUNK)])
        pltpu.sync_copy(zbuf.at[pl.ds(0, ACC_PT - 512)],
                        acc_sh.at[pl.ds(s * ACC_PT + 512, ACC_PT - 512)])

        @pl.when(s == 0)
        def _():
            pltpu.sync_copy(zbuf.at[pl.ds(0, YROWS - HALF)],
                            y_sh.at[pl.ds(HALF, YROWS - HALF)])

        pltpu.sync_copy(src_hbm.at[wid], src_all)
        pltpu.sync_copy(dst_hbm.at[wid], dst_all)

        for p in range(2):
            pbase = p * HALF
            @pl.when(s < 8)
            def _():
                pltpu.sync_copy(y_hbm.at[pl.ds(pbase + s * 625, 625)],
                                y_sh.at[pl.ds(s * 625, 625)])
            plsc.subcore_barrier()

            for g0 in range(GLAG):
                prep(g0, g0 % NBUF, pbase)
                gather(g0 % NBUF).start()

            @pl.loop(0, CPT // NBUF)
            def _(i):
                for b in range(NBUF):
                    g = i * NBUF + b

                    @pl.when(g >= SLAG)
                    def _():
                        scatter_wait((b - SLAG) % NBUF)

                    @pl.when(g + GLAG < CPT)
                    def _():
                        bn = (b + GLAG) % NBUF
                        prep(g + GLAG, bn, pbase)
                        gather(bn).start()

                    gather(b).wait()
                    scatter_start(b)

            for g in range(CPT - SLAG, CPT):
                scatter_wait(g % NBUF)
            plsc.subcore_barrier()

        pltpu.sync_copy(acc_sh.at[pl.ds(s * ACC_PT, ACC_PT)],
                        out_hbm.at[c, pl.ds(s * ACC_PT, ACC_PT)])

    return prop_kernel


_sc_propagate = _make_propagate()


RB = 1280
GRID = NP // RB


def _dinv_of(degp_blk):
    deg = degp_blk[0, :] + degp_blk[1, :] + 1.0
    return lax.rsqrt(deg)


def _combine(p_ref):
    ssum = p_ref[0, :N] + p_ref[1, :N]
    return jnp.concatenate(
        [ssum, jnp.zeros((NP - N, HID), jnp.float32)], axis=0)


def _tc_matmul1(x_pad, w1):
    def body(x_ref, w_ref, z_ref):
        z_ref[...] = jnp.dot(x_ref[...], w_ref[...],
                             preferred_element_type=jnp.float32)

    return pl.pallas_call(
        body,
        grid=(GRID,),
        in_specs=[
            pl.BlockSpec((RB, D_IN), lambda i: (i, 0)),
            pl.BlockSpec((D_IN, HID), lambda i: (0, 0)),
        ],
        out_specs=pl.BlockSpec((RB, HID), lambda i: (i, 0)),
        out_shape=jax.ShapeDtypeStruct((NP, HID), jnp.float32),
    )(x_pad, w1)


def _tc_scale(z1, degp):
    def body(z_ref, degp_ref, y_ref):
        dinv = _dinv_of(degp_ref)
        y_ref[...] = z_ref[...] * dinv[:, None]

    return pl.pallas_call(
        body,
        out_shape=jax.ShapeDtypeStruct((NP, HID), jnp.float32),
    )(z1, degp)


def _tc_mid(p, y_prev, degp, b, w_next):
    def body(p_ref, y_ref, degp_ref, b_ref, w_ref, o_ref):
        dinv = _dinv_of(degp_ref)
        ssum = _combine(p_ref) + y_ref[...]
        h = jnp.maximum(ssum * dinv[:, None] + b_ref[...][None, :], 0.0)
        y = jnp.dot(h, w_ref[...], preferred_element_type=jnp.float32)
        o_ref[...] = y * dinv[:, None]

    return pl.pallas_call(
        body,
        out_shape=jax.ShapeDtypeStruct((NP, HID), jnp.float32),
    )(p, y_prev, degp, b, w_next)


def _tc_head(p, y_prev, degp, b3, h1, hb1, h2, hb2):
    def body(p_ref, y_ref, degp_ref, b_ref, h1_ref, hb1_ref, h2_ref, hb2_ref,
             o_ref):
        dinv = _dinv_of(degp_ref)
        ssum = _combine(p_ref) + y_ref[...]
        h = jnp.maximum(ssum * dinv[:, None] + b_ref[...][None, :], 0.0)
        g = jnp.dot(h, h1_ref[...], preferred_element_type=jnp.float32)
        g = jnp.maximum(g + hb1_ref[...][None, :], 0.0)
        o = jnp.dot(g, h2_ref[...], preferred_element_type=jnp.float32)
        o_ref[...] = o + hb2_ref[...][None, :]

    return pl.pallas_call(
        body,
        out_shape=jax.ShapeDtypeStruct((NP, 2), jnp.float32),
    )(p, y_prev, degp, b3, h1, hb1, h2, hb2)


def kernel(x, edge_index, W1, b1, W2, b2, W3, b3, H1, hb1, H2, hb2):
    ei = edge_index.astype(jnp.int32)
    src3 = jnp.concatenate(
        [ei[0], jnp.zeros((E_PAD - E,), jnp.int32)]).reshape(32, CPT * CHUNK)
    dst3 = jnp.concatenate(
        [ei[1], jnp.full((E_PAD - E,), JUNK, jnp.int32)]
    ).reshape(32, CPT, CHUNK)
    x_pad = jnp.pad(x.astype(jnp.float32), ((0, NP - N), (0, 0)))

    degp = _sc_degree(dst3)
    z1 = _tc_matmul1(x_pad, W1)
    y1 = _tc_scale(z1, degp)
    p1 = _sc_propagate(y1, src3, dst3)
    y2 = _tc_mid(p1, y1, degp, b1, W2)
    p2 = _sc_propagate(y2, src3, dst3)
    y3 = _tc_mid(p2, y2, degp, b2, W3)
    p3 = _sc_propagate(y3, src3, dst3)
    out = _tc_head(p3, y3, degp, b3, H1, hb1, H2, hb2)
    return out[:N]
